# baseline (device time: 18669 ns/iter reference)
import jax
import jax.numpy as jnp
from jax import lax
from jax.experimental import pallas as pl
from jax.experimental.pallas import tpu as pltpu

N_DEV = 4


def kernel(x, Wq, Wo, K_ext, V_ext):
    B, Sq, D = x.shape
    _, Skv, H, Dh = K_ext.shape
    HD = H * Dh
    ML = 128
    W = HD + ML

    x2 = x.reshape(B * Sq, D)
    k2 = K_ext.reshape(B * Skv, HD)
    v2 = V_ext.reshape(B * Skv, HD)

    def body(x_hbm, wq_hbm, wo_hbm, k_hbm, v_hbm, out_ref,
             xs, wqs, wos, ks, vs, buf, in_sems, send_sems, recv_sems):
        my = lax.axis_index("i")

        cp_x = pltpu.make_async_copy(x_hbm, xs, in_sems.at[0])
        cp_wq = pltpu.make_async_copy(wq_hbm, wqs, in_sems.at[1])
        cp_k = pltpu.make_async_copy(k_hbm, ks, in_sems.at[2])
        cp_v = pltpu.make_async_copy(v_hbm, vs, in_sems.at[3])
        cp_wo = pltpu.make_async_copy(wo_hbm, wos, in_sems.at[4])
        for cp in (cp_x, cp_wq, cp_k, cp_v, cp_wo):
            cp.start()

        barrier_sem = pltpu.get_barrier_semaphore()
        for d in (1, 2, 3):
            pl.semaphore_signal(
                barrier_sem, inc=1,
                device_id=((my + d) % N_DEV,),
                device_id_type=pl.DeviceIdType.MESH,
            )

        cp_x.wait()
        cp_wq.wait()
        xb = xs[...].astype(jnp.bfloat16)
        wqb = wqs[...].astype(jnp.bfloat16)
        q = lax.dot(xb, wqb, preferred_element_type=jnp.float32) * 0.125
        qb = q.astype(jnp.bfloat16)

        buf[0, :, HD:] = jnp.zeros((B * Sq, ML), jnp.bfloat16)
        rows = lax.broadcasted_iota(jnp.int32, (B * Sq, B * Skv), 0)
        cols = lax.broadcasted_iota(jnp.int32, (B * Sq, B * Skv), 1)
        mask = (rows // Sq == cols // Skv).astype(jnp.float32)
        cp_k.wait()
        cp_v.wait()
        kb = ks[...].astype(jnp.bfloat16)
        vb = vs[...].astype(jnp.bfloat16)
        o_blocks = []
        l_blocks = []
        for h in range(H):
            qh = qb[:, h * Dh:(h + 1) * Dh]
            kh = kb[:, h * Dh:(h + 1) * Dh]
            s = lax.dot_general(
                qh, kh, (((1,), (1,)), ((), ())),
                preferred_element_type=jnp.float32)
            p = jnp.exp(s) * mask
            l_blocks.append(
                jnp.sum(p, axis=1, keepdims=True).astype(jnp.bfloat16))
            o = lax.dot(p.astype(jnp.bfloat16), vb[:, h * Dh:(h + 1) * Dh],
                        preferred_element_type=jnp.float32)
            o_blocks.append(o.astype(jnp.bfloat16))
        buf[0, :, 0:HD] = jnp.concatenate(o_blocks, axis=1)
        buf[0, :, HD:HD + H] = jnp.concatenate(l_blocks, axis=1)

        pl.semaphore_wait(barrier_sem, 3)

        rdmas = []
        for d in (1, 2, 3):
            rdma = pltpu.make_async_remote_copy(
                src_ref=buf.at[0],
                dst_ref=buf.at[N_DEV - d],
                send_sem=send_sems.at[d - 1],
                recv_sem=recv_sems.at[N_DEV - d],
                device_id=((my + d) % N_DEV,),
                device_id_type=pl.DeviceIdType.MESH,
            )
            rdma.start()
            rdmas.append(rdma)
        for rdma in rdmas:
            rdma.wait()

        total = (buf[0].astype(jnp.float32) + buf[1].astype(jnp.float32)
                 + buf[2].astype(jnp.float32) + buf[3].astype(jnp.float32))
        linv = 1.0 / total[:, HD:HD + H]
        o_norm = []
        for h in range(H):
            oh = total[:, h * Dh:(h + 1) * Dh]
            o_norm.append((oh * linv[:, h:h + 1]).astype(jnp.bfloat16))
        attn = jnp.concatenate(o_norm, axis=1)
        cp_wo.wait()
        wob = wos[...].astype(jnp.bfloat16)
        out_ref[...] = lax.dot(attn, wob, preferred_element_type=jnp.float32)

    out2 = pl.pallas_call(
        body,
        out_shape=jax.ShapeDtypeStruct((B * Sq, D), jnp.float32),
        in_specs=[pl.BlockSpec(memory_space=pl.ANY)] * 5,
        out_specs=pl.BlockSpec(memory_space=pltpu.VMEM),
        scratch_shapes=[
            pltpu.VMEM((B * Sq, D), jnp.float32),
            pltpu.VMEM((D, D), jnp.float32),
            pltpu.VMEM((D, D), jnp.float32),
            pltpu.VMEM((B * Skv, HD), jnp.float32),
            pltpu.VMEM((B * Skv, HD), jnp.float32),
            pltpu.VMEM((N_DEV, B * Sq, W), jnp.bfloat16),
            pltpu.SemaphoreType.DMA((5,)),
            pltpu.SemaphoreType.DMA((3,)),
            pltpu.SemaphoreType.DMA((N_DEV,)),
        ],
        compiler_params=pltpu.CompilerParams(collective_id=0),
    )(x2, Wq, Wo, k2, v2)
    return out2.reshape(B, Sq, D)


# device time: 16671 ns/iter; 1.1198x vs baseline; 1.1198x over previous
import jax
import jax.numpy as jnp
from jax import lax
from jax.experimental import pallas as pl
from jax.experimental.pallas import tpu as pltpu

N_DEV = 4


def kernel(x, Wq, Wo, K_ext, V_ext):
    B, Sq, D = x.shape
    _, Skv, H, Dh = K_ext.shape
    HD = H * Dh
    ML = 128
    W = HD + ML
    R = B * Sq
    QR = R // N_DEV

    x2 = x.reshape(R, D)
    k2 = K_ext.reshape(B * Skv, HD)
    v2 = V_ext.reshape(B * Skv, HD)

    def body(x_hbm, wq_hbm, wo_hbm, k_hbm, v_hbm, out_ref,
             xs, wqs, wos, ks, vs, loc, pbuf, osend, obuf,
             in_sems, s1, r1, s2, r2):
        my = lax.axis_index("i")

        cp_x = pltpu.make_async_copy(x_hbm, xs, in_sems.at[0])
        cp_wq = pltpu.make_async_copy(wq_hbm, wqs, in_sems.at[1])
        cp_k = pltpu.make_async_copy(k_hbm, ks, in_sems.at[2])
        cp_v = pltpu.make_async_copy(v_hbm, vs, in_sems.at[3])
        cp_wo = pltpu.make_async_copy(wo_hbm, wos, in_sems.at[4])
        for cp in (cp_x, cp_wq, cp_k, cp_v, cp_wo):
            cp.start()

        barrier_sem = pltpu.get_barrier_semaphore()
        for d in (1, 2, 3):
            pl.semaphore_signal(
                barrier_sem, inc=1,
                device_id=((my + d) % N_DEV,),
                device_id_type=pl.DeviceIdType.MESH,
            )

        cp_x.wait()
        cp_wq.wait()
        xb = xs[...].astype(jnp.bfloat16)
        wqb = wqs[...].astype(jnp.bfloat16)
        q = lax.dot(xb, wqb, preferred_element_type=jnp.float32) * 0.125
        qb = q.astype(jnp.bfloat16)

        loc[:, HD:] = jnp.zeros((R, ML), jnp.bfloat16)
        rows = lax.broadcasted_iota(jnp.int32, (R, B * Skv), 0)
        cols = lax.broadcasted_iota(jnp.int32, (R, B * Skv), 1)
        mask = (rows // Sq == cols // Skv).astype(jnp.float32)
        cp_k.wait()
        cp_v.wait()
        kb = ks[...].astype(jnp.bfloat16)
        vb = vs[...].astype(jnp.bfloat16)
        o_blocks = []
        l_blocks = []
        for h in range(H):
            qh = qb[:, h * Dh:(h + 1) * Dh]
            kh = kb[:, h * Dh:(h + 1) * Dh]
            s = lax.dot_general(
                qh, kh, (((1,), (1,)), ((), ())),
                preferred_element_type=jnp.float32)
            p = jnp.exp(s) * mask
            l_blocks.append(
                jnp.sum(p, axis=1, keepdims=True).astype(jnp.bfloat16))
            o = lax.dot(p.astype(jnp.bfloat16), vb[:, h * Dh:(h + 1) * Dh],
                        preferred_element_type=jnp.float32)
            o_blocks.append(o.astype(jnp.bfloat16))
        loc[:, 0:HD] = jnp.concatenate(o_blocks, axis=1)
        loc[:, HD:HD + H] = jnp.concatenate(l_blocks, axis=1)

        pl.semaphore_wait(barrier_sem, 3)

        rdmas1 = []
        for d in (1, 2, 3):
            t = (my + d) % N_DEV
            rdma = pltpu.make_async_remote_copy(
                src_ref=loc.at[pl.ds(t * QR, QR)],
                dst_ref=pbuf.at[N_DEV - d],
                send_sem=s1.at[d - 1],
                recv_sem=r1.at[N_DEV - d],
                device_id=(t,),
                device_id_type=pl.DeviceIdType.MESH,
            )
            rdma.start()
            rdmas1.append(rdma)
        for rdma in rdmas1:
            rdma.wait()

        mine = loc[pl.ds(my * QR, QR), :].astype(jnp.float32)
        total = (mine + pbuf[1].astype(jnp.float32)
                 + pbuf[2].astype(jnp.float32) + pbuf[3].astype(jnp.float32))
        linv = 1.0 / total[:, HD:HD + H]
        o_norm = []
        for h in range(H):
            oh = total[:, h * Dh:(h + 1) * Dh]
            o_norm.append((oh * linv[:, h:h + 1]).astype(jnp.bfloat16))
        attn = jnp.concatenate(o_norm, axis=1)
        cp_wo.wait()
        wob = wos[...].astype(jnp.bfloat16)
        outq = lax.dot(attn, wob, preferred_element_type=jnp.float32)
        out_ref[pl.ds(my * QR, QR), :] = outq
        osend[...] = outq.astype(jnp.bfloat16)

        rdmas2 = []
        for d in (1, 2, 3):
            rdma = pltpu.make_async_remote_copy(
                src_ref=osend,
                dst_ref=obuf.at[N_DEV - d],
                send_sem=s2.at[d - 1],
                recv_sem=r2.at[N_DEV - d],
                device_id=((my + d) % N_DEV,),
                device_id_type=pl.DeviceIdType.MESH,
            )
            rdma.start()
            rdmas2.append(rdma)
        for rdma in rdmas2:
            rdma.wait()
        for j in (1, 2, 3):
            src_dev = (my + j) % N_DEV
            out_ref[pl.ds(src_dev * QR, QR), :] = obuf[j].astype(jnp.float32)

    out2 = pl.pallas_call(
        body,
        out_shape=jax.ShapeDtypeStruct((R, D), jnp.float32),
        in_specs=[pl.BlockSpec(memory_space=pl.ANY)] * 5,
        out_specs=pl.BlockSpec(memory_space=pltpu.VMEM),
        scratch_shapes=[
            pltpu.VMEM((R, D), jnp.float32),
            pltpu.VMEM((D, D), jnp.float32),
            pltpu.VMEM((D, D), jnp.float32),
            pltpu.VMEM((B * Skv, HD), jnp.float32),
            pltpu.VMEM((B * Skv, HD), jnp.float32),
            pltpu.VMEM((R, W), jnp.bfloat16),
            pltpu.VMEM((N_DEV, QR, W), jnp.bfloat16),
            pltpu.VMEM((QR, HD), jnp.bfloat16),
            pltpu.VMEM((N_DEV, QR, HD), jnp.bfloat16),
            pltpu.SemaphoreType.DMA((5,)),
            pltpu.SemaphoreType.DMA((3,)),
            pltpu.SemaphoreType.DMA((N_DEV,)),
            pltpu.SemaphoreType.DMA((3,)),
            pltpu.SemaphoreType.DMA((N_DEV,)),
        ],
        compiler_params=pltpu.CompilerParams(collective_id=0),
    )(x2, Wq, Wo, k2, v2)
    return out2.reshape(B, Sq, D)
